# baseline (speedup 1.0000x reference)
import functools
import math

import jax
import jax.numpy as jnp
from jax import lax
from jax.experimental import pallas as pl
from jax.experimental.pallas import tpu as pltpu
from jax.experimental.pallas import tpu_sc as plsc

VOCAB = 1000000
D = 64
SCALE = math.sqrt(float(D))

NC = 2
NS = 16
NW = NC * NS
L = 16

N_FULL = VOCAB // 128
S_MAIN = N_FULL // NW
N_EXTRA = N_FULL - S_MAIN * NW
TAIL = VOCAB - N_FULL * 128
TAIL_BASE = N_FULL * 128

_mesh = plsc.VectorSubcoreMesh(core_axis_name="c", subcore_axis_name="s")


@functools.partial(
    pl.kernel,
    mesh=_mesh,
    out_type=jax.ShapeDtypeStruct((VOCAB * D,), jnp.float32),
    scratch_types=[
        pltpu.VMEM((D, 128), jnp.float32),
        pltpu.VMEM((D, 128), jnp.float32),
        pltpu.VMEM((128 * D,), jnp.float32),
        pltpu.VMEM((128 * D,), jnp.float32),
        pltpu.SemaphoreType.DMA,
        pltpu.SemaphoreType.DMA,
        pltpu.SemaphoreType.DMA,
        pltpu.SemaphoreType.DMA,
    ],
    compiler_params=pltpu.CompilerParams(use_tc_tiling_on_sc=True,
                                         needs_layout_passes=False),
)
def _transpose_scale(tt_hbm, tail_hbm, out_hbm,
                     blk0, blk1, row0, row1, si0, si1, so0, so1):
    wid = lax.axis_index("s") * NC + lax.axis_index("c")
    blk = (blk0, blk1)
    row = (row0, row1)
    si = (si0, si1)
    so = (so0, so1)
    iota = lax.iota(jnp.int32, L)
    iota64 = iota * D
    perms = [(iota + s) & 15 for s in range(16)]
    qs = [iota64 + p for p in perms]

    def chunk_id(s):
        return s * NW + wid

    def in_start(c, b):
        pltpu.async_copy(tt_hbm.at[:, pl.ds(c * 128, 128)], blk[b], si[b])

    def in_wait(b):
        pltpu.make_async_copy(
            tt_hbm.at[:, pl.ds(0, 128)], blk[b], si[b]).wait()

    def out_start(c, b):
        pltpu.async_copy(row[b], out_hbm.at[pl.ds(c * 128 * D, 128 * D)],
                         so[b])

    def out_wait(b):
        pltpu.make_async_copy(
            row[b], out_hbm.at[pl.ds(0, 128 * D)], so[b]).wait()

    def transpose_scale(b):
        src, dst = blk[b], row[b]

        @plsc.parallel_loop(0, 8, unroll=2)
        def body(i0b):
            i0 = i0b * 16
            cols = iota + i0
            for d0 in (0, 16, 32, 48):
                for s in range(16):
                    v = plsc.load_gather(src, [perms[s] + d0, cols]) * SCALE
                    plsc.store_scatter(dst, [qs[s] + (i0 * D + d0)], v)

    in_start(chunk_id(0), 0)

    def grp(gi, carry):
        for b in (0, 1):
            s = 2 * gi + b
            in_wait(b)

            @pl.when(s < S_MAIN - 1)
            def _():
                in_start(chunk_id(s + 1), 1 - b)

            @pl.when(s >= 2)
            def _():
                out_wait(b)

            transpose_scale(b)
            out_start(chunk_id(s), b)
        return carry

    lax.fori_loop(0, S_MAIN // 2, grp, 0)
    out_wait(0)
    out_wait(1)

    @pl.when(wid < N_EXTRA)
    def _():
        c = S_MAIN * NW + wid
        pltpu.sync_copy(tt_hbm.at[:, pl.ds(c * 128, 128)], blk0)
        transpose_scale(0)
        pltpu.sync_copy(row0, out_hbm.at[pl.ds(c * 128 * D, 128 * D)])

    @pl.when(wid == NW - 1)
    def _():
        pltpu.sync_copy(tail_hbm, row1.at[pl.ds(0, TAIL * D)])

        def tail_body(i, carry):
            sl = pl.ds(i * L, L)
            row1[sl] = row1[sl] * SCALE
            return carry

        lax.fori_loop(0, TAIL * D // L, tail_body, 0)
        pltpu.sync_copy(row1.at[pl.ds(0, TAIL * D)],
                        out_hbm.at[pl.ds(TAIL_BASE * D, TAIL * D)])


B_TOT = 4096 * 200
SLABS = B_TOT // 128
SLABS_W = SLABS // NW
CH = 256
K_CH = SLABS_W // 2


@functools.partial(
    pl.kernel,
    mesh=_mesh,
    out_type=jax.ShapeDtypeStruct((B_TOT * D,), jnp.float32),
    scratch_types=[
        pltpu.VMEM((SLABS_W * 128,), jnp.int32),
        pltpu.VMEM((CH, D), jnp.float32),
        pltpu.VMEM((CH, D), jnp.float32),
        pltpu.VMEM((CH * D,), jnp.float32),
        pltpu.VMEM((CH * D,), jnp.float32),
        pltpu.SemaphoreType.DMA,
        pltpu.SemaphoreType.DMA,
        pltpu.SemaphoreType.DMA,
        pltpu.SemaphoreType.DMA,
    ],
    compiler_params=pltpu.CompilerParams(use_tc_tiling_on_sc=False,
                                         needs_layout_passes=False),
)
def _gather_native(idx_hbm, tbl_hbm, out_hbm,
                   idx_all, g0, g1, t0, t1, sg0, sg1, so0, so1):
    wid = lax.axis_index("s") * NC + lax.axis_index("c")
    g = (g0, g1)
    tb = (t0, t1)
    sg = (sg0, sg1)
    so = (so0, so1)
    iota = lax.iota(jnp.int32, L)
    perms = [(iota + s) & 15 for s in range(16)]
    ws = [(p >> 3) * 2048 + (p & 7) * 128 + iota for p in perms]

    pltpu.sync_copy(idx_hbm.at[pl.ds(wid * SLABS_W * 128, SLABS_W * 128)],
                    idx_all)

    def gather_start(k, b):
        pltpu.async_copy(
            tbl_hbm.at[idx_all.at[pl.ds(k * CH, CH)]], g[b], sg[b])

    def gather_wait(b):
        pltpu.make_async_copy(tbl_hbm.at[pl.ds(0, CH)], g[b], sg[b]).wait()

    def outs_start(k, b):
        slab = wid * SLABS_W + k * 2
        o0 = slab * 1024 + (slab >> 5) * (7 * 32768)
        for d_blk in range(8):
            pltpu.async_copy(tb[b].at[pl.ds(d_blk * 2048, 2048)],
                             out_hbm.at[pl.ds(o0 + d_blk * 32768, 2048)],
                             so[b])

    def outs_wait(b):
        pltpu.make_async_copy(
            out_hbm.at[pl.ds(0, CH * D)], tb[b], so[b]).wait()

    def transpose(b):
        src, dst = g[b], tb[b]

        @plsc.parallel_loop(0, CH // L, unroll=1)
        def body(q):
            r0 = q * L
            off0 = (r0 >> 7) * 1024 + (r0 & 127)
            rows = iota + r0
            for h in range(4):
                for s in range(16):
                    v = plsc.load_gather(src, [rows, perms[s] + h * 16])
                    plsc.store_scatter(dst, [ws[s] + (off0 + 4096 * h)], v)

    gather_start(0, 0)

    def grp(gi, carry):
        for b in (0, 1):
            k = 2 * gi + b
            gather_wait(b)

            @pl.when(k < K_CH - 1)
            def _():
                gather_start(k + 1, 1 - b)

            @pl.when(k >= 2)
            def _():
                outs_wait(b)

            transpose(b)
            outs_start(k, b)
        return carry

    lax.fori_loop(0, K_CH // 2, grp, 0)
    outs_wait(0)
    outs_wait(1)


def kernel(x, table):
    tt = table.T
    tail = lax.slice(table, (TAIL_BASE, 0), (VOCAB, D)).reshape(-1)
    flat = _transpose_scale(tt, tail)
    tbl = flat.reshape(VOCAB, D)
    idx = x.T.reshape(-1).astype(jnp.int32)
    o = _gather_native(idx, tbl)
    o5 = o.reshape(200, 8, 32, 8, 128)
    return o5.transpose(2, 4, 0, 1, 3).reshape(4096, 200, D)


# speedup vs baseline: 2.0285x; 2.0285x over previous
import functools
import math

import jax
import jax.numpy as jnp
from jax import lax
from jax.experimental import pallas as pl
from jax.experimental.pallas import tpu as pltpu
from jax.experimental.pallas import tpu_sc as plsc

VOCAB = 1000000
D = 64
SCALE = math.sqrt(float(D))

NC = 2
NS = 16
NW = NC * NS
L = 16

N_FULL = VOCAB // 128
S_MAIN = N_FULL // NW
N_EXTRA = N_FULL - S_MAIN * NW
TAIL = VOCAB - N_FULL * 128
TAIL_BASE = N_FULL * 128

_mesh = plsc.VectorSubcoreMesh(core_axis_name="c", subcore_axis_name="s")


@functools.partial(
    pl.kernel,
    mesh=_mesh,
    out_type=jax.ShapeDtypeStruct((VOCAB * D,), jnp.float32),
    scratch_types=[
        pltpu.VMEM((D, 128), jnp.float32),
        pltpu.VMEM((D, 128), jnp.float32),
        pltpu.VMEM((128 * D,), jnp.float32),
        pltpu.VMEM((128 * D,), jnp.float32),
        pltpu.SemaphoreType.DMA,
        pltpu.SemaphoreType.DMA,
        pltpu.SemaphoreType.DMA,
        pltpu.SemaphoreType.DMA,
    ],
    compiler_params=pltpu.CompilerParams(use_tc_tiling_on_sc=True,
                                         needs_layout_passes=False),
)
def _transpose_scale(tt_hbm, tail_hbm, out_hbm,
                     blk0, blk1, row0, row1, si0, si1, so0, so1):
    wid = lax.axis_index("s") * NC + lax.axis_index("c")
    blk = (blk0, blk1)
    row = (row0, row1)
    si = (si0, si1)
    so = (so0, so1)
    iota = lax.iota(jnp.int32, L)
    iota64 = iota * D
    perms = [(iota + s) & 15 for s in range(16)]
    qs = [iota64 + p for p in perms]

    def chunk_id(s):
        return s * NW + wid

    def in_start(c, b):
        pltpu.async_copy(tt_hbm.at[:, pl.ds(c * 128, 128)], blk[b], si[b])

    def in_wait(b):
        pltpu.make_async_copy(
            tt_hbm.at[:, pl.ds(0, 128)], blk[b], si[b]).wait()

    def out_start(c, b):
        pltpu.async_copy(row[b], out_hbm.at[pl.ds(c * 128 * D, 128 * D)],
                         so[b])

    def out_wait(b):
        pltpu.make_async_copy(
            row[b], out_hbm.at[pl.ds(0, 128 * D)], so[b]).wait()

    def transpose_scale(b):
        src, dst = blk[b], row[b]

        @plsc.parallel_loop(0, 32, unroll=1)
        def body(q):
            i0 = (q >> 2) * 16
            d0 = (q & 3) * 16
            cols = iota + i0
            base = i0 * D + d0
            for s in range(16):
                rows = perms[s] + d0
                v = plsc.load_gather(src, [rows, cols]) * SCALE
                plsc.store_scatter(dst, [iota64 + (rows - d0 + base)], v)

    in_start(chunk_id(0), 0)

    def grp(gi, carry):
        for b in (0, 1):
            s = 2 * gi + b
            in_wait(b)

            @pl.when(s < S_MAIN - 1)
            def _():
                in_start(chunk_id(s + 1), 1 - b)

            @pl.when(s >= 2)
            def _():
                out_wait(b)

            transpose_scale(b)
            out_start(chunk_id(s), b)
        return carry

    lax.fori_loop(0, S_MAIN // 2, grp, 0)
    out_wait(0)
    out_wait(1)

    @pl.when(wid < N_EXTRA)
    def _():
        c = S_MAIN * NW + wid
        pltpu.sync_copy(tt_hbm.at[:, pl.ds(c * 128, 128)], blk0)
        transpose_scale(0)
        pltpu.sync_copy(row0, out_hbm.at[pl.ds(c * 128 * D, 128 * D)])

    @pl.when(wid == NW - 1)
    def _():
        pltpu.sync_copy(tail_hbm, row1.at[pl.ds(0, TAIL * D)])

        def tail_body(i, carry):
            sl = pl.ds(i * L, L)
            row1[sl] = row1[sl] * SCALE
            return carry

        lax.fori_loop(0, TAIL * D // L, tail_body, 0)
        pltpu.sync_copy(row1.at[pl.ds(0, TAIL * D)],
                        out_hbm.at[pl.ds(TAIL_BASE * D, TAIL * D)])


B_TOT = 4096 * 200
SLABS = B_TOT // 128
SLABS_W = SLABS // NW
CH = 256
K_CH = SLABS_W // 2


@functools.partial(
    pl.kernel,
    mesh=_mesh,
    out_type=jax.ShapeDtypeStruct((B_TOT * D,), jnp.float32),
    scratch_types=[
        pltpu.VMEM((SLABS_W * 128,), jnp.int32),
        pltpu.VMEM((CH, D), jnp.float32),
        pltpu.VMEM((CH, D), jnp.float32),
        pltpu.VMEM((CH * D,), jnp.float32),
        pltpu.VMEM((CH * D,), jnp.float32),
        pltpu.SemaphoreType.DMA,
        pltpu.SemaphoreType.DMA,
        pltpu.SemaphoreType.DMA,
        pltpu.SemaphoreType.DMA,
    ],
    compiler_params=pltpu.CompilerParams(use_tc_tiling_on_sc=False,
                                         needs_layout_passes=False),
)
def _gather_native(idx_hbm, tbl_hbm, out_hbm,
                   idx_all, g0, g1, t0, t1, sg0, sg1, so0, so1):
    wid = lax.axis_index("s") * NC + lax.axis_index("c")
    g = (g0, g1)
    tb = (t0, t1)
    sg = (sg0, sg1)
    so = (so0, so1)
    iota = lax.iota(jnp.int32, L)
    perms = [(iota + s) & 15 for s in range(16)]
    ws = [(p >> 3) * 2048 + (p & 7) * 128 + iota for p in perms]

    pltpu.sync_copy(idx_hbm.at[pl.ds(wid * SLABS_W * 128, SLABS_W * 128)],
                    idx_all)

    def gather_start(k, b):
        pltpu.async_copy(
            tbl_hbm.at[idx_all.at[pl.ds(k * CH, CH)]], g[b], sg[b])

    def gather_wait(b):
        pltpu.make_async_copy(tbl_hbm.at[pl.ds(0, CH)], g[b], sg[b]).wait()

    def outs_start(k, b):
        slab = wid * SLABS_W + k * 2
        o0 = slab * 1024 + (slab >> 5) * (7 * 32768)
        for d_blk in range(8):
            pltpu.async_copy(tb[b].at[pl.ds(d_blk * 2048, 2048)],
                             out_hbm.at[pl.ds(o0 + d_blk * 32768, 2048)],
                             so[b])

    def outs_wait(b):
        pltpu.make_async_copy(
            out_hbm.at[pl.ds(0, CH * D)], tb[b], so[b]).wait()

    def transpose(b):
        src, dst = g[b], tb[b]

        @plsc.parallel_loop(0, CH // L, unroll=1)
        def body(q):
            r0 = q * L
            off0 = (r0 >> 7) * 1024 + (r0 & 127)
            rows = iota + r0
            for h in range(4):
                for s in range(16):
                    v = plsc.load_gather(src, [rows, perms[s] + h * 16])
                    plsc.store_scatter(dst, [ws[s] + (off0 + 4096 * h)], v)

    gather_start(0, 0)

    def grp(gi, carry):
        for b in (0, 1):
            k = 2 * gi + b
            gather_wait(b)

            @pl.when(k < K_CH - 1)
            def _():
                gather_start(k + 1, 1 - b)

            @pl.when(k >= 2)
            def _():
                outs_wait(b)

            transpose(b)
            outs_start(k, b)
        return carry

    lax.fori_loop(0, K_CH // 2, grp, 0)
    outs_wait(0)
    outs_wait(1)


def kernel(x, table):
    tt = table.T
    tail = lax.slice(table, (TAIL_BASE, 0), (VOCAB, D)).reshape(-1)
    flat = _transpose_scale(tt, tail)
    tbl = flat.reshape(VOCAB, D)
    idx = x.T.reshape(-1).astype(jnp.int32)
    o = _gather_native(idx, tbl)
    o5 = o.reshape(200, 8, 32, 8, 128)
    return o5.transpose(2, 4, 0, 1, 3).reshape(4096, 200, D)
